# baseline (speedup 1.0000x reference)
import numpy as np
import jax
import jax.numpy as jnp
from jax.experimental import pallas as pl
from jax.experimental.pallas import tpu as pltpu

_RES = 128
_SIG = 10.0
_ROWS = 8320
_RCHUNK = 320


def _spec_consts():
    freqs = [np.fft.fftfreq(_RES, d=1.0 / _RES)] * 2
    freqs.append(np.fft.rfftfreq(_RES, d=1.0 / _RES))
    om = np.stack(np.meshgrid(*freqs, indexing="ij"), axis=-1)
    dis = np.sqrt((om ** 2).sum(-1))
    g = np.exp(-0.5 * ((_SIG * 2.0 * dis / _RES) ** 2))
    lap = -np.sum((2.0 * np.pi * om) ** 2, axis=-1)
    c = 2.0 * np.pi * g / (lap + 1e-6)
    b = np.moveaxis(om, -1, 0) * c
    return b.astype(np.float32).reshape(3, _ROWS, 128)


_B_CONST = _spec_consts()


def _spectral_combine(Fr, Fi):
    B = jnp.asarray(_B_CONST)

    def body(fr_ref, fi_ref, b_ref, o_ref):
        b0, b1, b2 = b_ref[0], b_ref[1], b_ref[2]
        for b in range(2):
            o_ref[2 * b] = (b0 * fi_ref[3 * b] + b1 * fi_ref[3 * b + 1]
                            + b2 * fi_ref[3 * b + 2])
            o_ref[2 * b + 1] = -(b0 * fr_ref[3 * b] + b1 * fr_ref[3 * b + 1]
                                 + b2 * fr_ref[3 * b + 2])

    return pl.pallas_call(
        body,
        grid=(_ROWS // _RCHUNK,),
        in_specs=[
            pl.BlockSpec((6, _RCHUNK, 128), lambda i: (0, i, 0)),
            pl.BlockSpec((6, _RCHUNK, 128), lambda i: (0, i, 0)),
            pl.BlockSpec((3, _RCHUNK, 128), lambda i: (0, i, 0)),
        ],
        out_specs=pl.BlockSpec((4, _RCHUNK, 128), lambda i: (0, i, 0)),
        out_shape=jax.ShapeDtypeStruct((4, _ROWS, 128), jnp.float32),
    )(Fr, Fi, B)


def _rasterize(V, N):
    bs, npts, _ = V.shape
    nf = N.shape[-1]
    size = jnp.float32(_RES)
    t = V * size
    ind0 = jnp.floor(t).astype(jnp.int32)
    ind1 = jnp.mod(jnp.ceil(t), size).astype(jnp.int32)
    f = t - ind0.astype(jnp.float32)
    com_np = np.stack(np.meshgrid(*([np.array([0, 1])] * 3), indexing="ij"),
                      axis=-1).reshape(-1, 3)
    com = jnp.asarray(com_np, dtype=jnp.int32)[None, None]
    ind_n = jnp.where(com == 0, ind0[:, :, None, :], ind1[:, :, None, :])
    w_ax = jnp.where(com == 0, (1.0 - f)[:, :, None, :], f[:, :, None, :])
    weights = jnp.prod(w_ax, axis=-1)
    vals_sc = weights[..., None] * N[:, :, None, :]
    lin = (ind_n[..., 0] * _RES + ind_n[..., 1]) * _RES + ind_n[..., 2]
    b_idx = jnp.arange(bs, dtype=jnp.int32)[:, None, None, None]
    f_idx = jnp.arange(nf, dtype=jnp.int32)[None, None, None, :]
    flat = (b_idx * nf + f_idx) * (_RES ** 3) + lin[..., None]
    out = jnp.zeros(bs * nf * _RES ** 3, dtype=N.dtype)
    out = out.at[flat.reshape(-1)].add(vals_sc.reshape(-1))
    return out.reshape(bs, nf, _RES, _RES, _RES)


def kernel(V, N):
    grid = _rasterize(V, N)
    F = jnp.fft.rfftn(grid, axes=(2, 3, 4))
    Fr = jnp.real(F).reshape(6, _ROWS, 128)
    Fi = jnp.imag(F).reshape(6, _ROWS, 128)
    O = _spectral_combine(Fr, Fi)
    Phi = O.reshape(2, 2, 128, 128, 65).transpose(2, 3, 4, 1, 0)
    return Phi.at[0, 0, 0].set(0.0)


# speedup vs baseline: 19.1822x; 19.1822x over previous
import functools

import numpy as np
import jax
import jax.numpy as jnp
from jax import lax
from jax.experimental import pallas as pl
from jax.experimental.pallas import tpu as pltpu
from jax.experimental.pallas import tpu_sc as plsc

_RES = 128
_SIG = 10.0
_ROWS = 8320
_RCHUNK = 320


def _spec_consts():
    freqs = [np.fft.fftfreq(_RES, d=1.0 / _RES)] * 2
    freqs.append(np.fft.rfftfreq(_RES, d=1.0 / _RES))
    om = np.stack(np.meshgrid(*freqs, indexing="ij"), axis=-1)
    dis = np.sqrt((om ** 2).sum(-1))
    g = np.exp(-0.5 * ((_SIG * 2.0 * dis / _RES) ** 2))
    lap = -np.sum((2.0 * np.pi * om) ** 2, axis=-1)
    c = 2.0 * np.pi * g / (lap + 1e-6)
    b = np.moveaxis(om, -1, 0) * c
    return b.astype(np.float32).reshape(3, _ROWS, 128)


_B_CONST = _spec_consts()


def _spectral_combine(Fr, Fi):
    B = jnp.asarray(_B_CONST)

    def body(fr_ref, fi_ref, b_ref, o_ref):
        b0, b1, b2 = b_ref[0], b_ref[1], b_ref[2]
        for b in range(2):
            o_ref[2 * b] = (b0 * fi_ref[3 * b] + b1 * fi_ref[3 * b + 1]
                            + b2 * fi_ref[3 * b + 2])
            o_ref[2 * b + 1] = -(b0 * fr_ref[3 * b] + b1 * fr_ref[3 * b + 1]
                                 + b2 * fr_ref[3 * b + 2])

    return pl.pallas_call(
        body,
        grid=(_ROWS // _RCHUNK,),
        in_specs=[
            pl.BlockSpec((6, _RCHUNK, 128), lambda i: (0, i, 0)),
            pl.BlockSpec((6, _RCHUNK, 128), lambda i: (0, i, 0)),
            pl.BlockSpec((3, _RCHUNK, 128), lambda i: (0, i, 0)),
        ],
        out_specs=pl.BlockSpec((4, _RCHUNK, 128), lambda i: (0, i, 0)),
        out_shape=jax.ShapeDtypeStruct((4, _ROWS, 128), jnp.float32),
    )(Fr, Fi, B)


_P = 100352
_TPTS = _P // 16
_CHUNK = 1568
_NROW = _CHUNK // 16
_SLABW = 64
_SLAB = _SLABW * _RES * _RES
_DUMP = _SLAB
_ACC = _SLAB + 16384
_STRIPE = _SLAB // 16


def _sc_rasterize(Vt, Nt):
    mesh = plsc.VectorSubcoreMesh(core_axis_name="c", subcore_axis_name="s")

    @functools.partial(
        pl.kernel,
        out_type=jax.ShapeDtypeStruct((12 * _SLAB,), jnp.float32),
        mesh=mesh,
        scratch_types=[
            pltpu.VMEM_SHARED((_ACC,), jnp.float32),
            pltpu.VMEM((2048,), jnp.float32),
            pltpu.VMEM((_CHUNK,), jnp.float32),
            pltpu.VMEM((_CHUNK,), jnp.float32),
            pltpu.VMEM((_CHUNK,), jnp.float32),
            pltpu.VMEM((_CHUNK,), jnp.float32),
            pltpu.VMEM((_NROW * 128,), jnp.int32),
            pltpu.VMEM((_NROW * 128,), jnp.float32),
        ],
    )
    def k(v_hbm, n_hbm, out_hbm, acc, zbuf, px, py, pz, nv, idxb, valb):
        c = lax.axis_index("c")
        s = lax.axis_index("s")

        def zinit(i, carry):
            zbuf[pl.ds(i * 16, 16)] = jnp.zeros((16,), jnp.float32)
            return carry

        lax.fori_loop(0, 2048 // 16, zinit, 0)

        def axis_terms(pref, base, slab):
            t = pref[pl.ds(base, 16)] * 128.0
            i0 = t.astype(jnp.int32)
            f = t - i0.astype(jnp.float32)
            i1 = jnp.where(f > 0.0, i0 + 1, i0) & 127
            return i0, i1, f, slab

        def job(j, carry):
            g = j * 2 + c
            b = g // 6
            f_feat = (g % 6) // 2
            slab = c
            bf = b * 3 + f_feat

            def zero(i, carry2):
                pltpu.sync_copy(
                    zbuf, acc.at[pl.ds(s * _STRIPE + i * 2048, 2048)])
                return carry2

            lax.fori_loop(0, _STRIPE // 2048, zero, 0)
            plsc.subcore_barrier()

            def chunk(cc, carry2):
                pbase = s * _TPTS + cc * _CHUNK
                vb = b * 3 * _P
                pltpu.sync_copy(v_hbm.at[pl.ds(vb + pbase, _CHUNK)], px)
                pltpu.sync_copy(v_hbm.at[pl.ds(vb + _P + pbase, _CHUNK)], py)
                pltpu.sync_copy(v_hbm.at[pl.ds(vb + 2 * _P + pbase, _CHUNK)], pz)
                pltpu.sync_copy(n_hbm.at[pl.ds((b * 3 + f_feat) * _P + pbase, _CHUNK)], nv)

                def row(i, carry3):
                    base = i * 16
                    tx = px[pl.ds(base, 16)] * 128.0
                    x0 = tx.astype(jnp.int32)
                    fx = tx - x0.astype(jnp.float32)
                    x1 = jnp.where(fx > 0.0, x0 + 1, x0) & 127
                    ty = py[pl.ds(base, 16)] * 128.0
                    y0 = ty.astype(jnp.int32)
                    fy = ty - y0.astype(jnp.float32)
                    y1 = jnp.where(fy > 0.0, y0 + 1, y0) & 127
                    tz = pz[pl.ds(base, 16)] * 128.0
                    z0 = tz.astype(jnp.int32)
                    fz = tz - z0.astype(jnp.float32)
                    z1 = jnp.where(fz > 0.0, z0 + 1, z0) & 127
                    val = nv[pl.ds(base, 16)]

                    dump = jnp.full((16,), _DUMP, jnp.int32)
                    xo0 = jnp.where((x0 >> 6) == slab, (x0 & 63) * 16384, dump)
                    xo1 = jnp.where((x1 >> 6) == slab, (x1 & 63) * 16384, dump)
                    a00 = xo0 + y0 * 128
                    a01 = xo0 + y1 * 128
                    a10 = xo1 + y0 * 128
                    a11 = xo1 + y1 * 128
                    wx0 = 1.0 - fx
                    wy0 = 1.0 - fy
                    wz0 = (1.0 - fz) * val
                    wz1 = fz * val
                    w00 = wx0 * wy0
                    w01 = wx0 * fy
                    w10 = fx * wy0
                    w11 = fx * fy
                    idxb[pl.ds(i * 128 + 0, 16)] = a00 + z0
                    valb[pl.ds(i * 128 + 0, 16)] = w00 * wz0
                    idxb[pl.ds(i * 128 + 16, 16)] = a00 + z1
                    valb[pl.ds(i * 128 + 16, 16)] = w00 * wz1
                    idxb[pl.ds(i * 128 + 32, 16)] = a01 + z0
                    valb[pl.ds(i * 128 + 32, 16)] = w01 * wz0
                    idxb[pl.ds(i * 128 + 48, 16)] = a01 + z1
                    valb[pl.ds(i * 128 + 48, 16)] = w01 * wz1
                    idxb[pl.ds(i * 128 + 64, 16)] = a10 + z0
                    valb[pl.ds(i * 128 + 64, 16)] = w10 * wz0
                    idxb[pl.ds(i * 128 + 80, 16)] = a10 + z1
                    valb[pl.ds(i * 128 + 80, 16)] = w10 * wz1
                    idxb[pl.ds(i * 128 + 96, 16)] = a11 + z0
                    valb[pl.ds(i * 128 + 96, 16)] = w11 * wz0
                    idxb[pl.ds(i * 128 + 112, 16)] = a11 + z1
                    valb[pl.ds(i * 128 + 112, 16)] = w11 * wz1
                    return carry3

                lax.fori_loop(0, _NROW, row, 0)
                pltpu.sync_copy(valb, acc.at[idxb], add=True)
                return carry2

            lax.fori_loop(0, _TPTS // _CHUNK, chunk, 0)
            plsc.subcore_barrier()

            pltpu.sync_copy(
                acc.at[pl.ds(s * _STRIPE, _STRIPE)],
                out_hbm.at[pl.ds(bf * 2 * _SLAB + slab * _SLAB + s * _STRIPE,
                                 _STRIPE)])
            return carry

        lax.fori_loop(0, 6, job, 0)

    return k(Vt, Nt)


def _rasterize(V, N):
    Vt = jnp.pad(jnp.transpose(V, (0, 2, 1)),
                 ((0, 0), (0, 0), (0, _P - V.shape[1]))).reshape(-1)
    Nt = jnp.pad(jnp.transpose(N, (0, 2, 1)),
                 ((0, 0), (0, 0), (0, _P - N.shape[1]))).reshape(-1)
    out = _sc_rasterize(Vt, Nt)
    return out.reshape(2, 3, _RES, _RES, _RES)


def kernel(V, N):
    grid = _rasterize(V, N)
    F = jnp.fft.rfftn(grid, axes=(2, 3, 4))
    Fr = jnp.real(F).reshape(6, _ROWS, 128)
    Fi = jnp.imag(F).reshape(6, _ROWS, 128)
    O = _spectral_combine(Fr, Fi)
    Phi = O.reshape(2, 2, 128, 128, 65).transpose(2, 3, 4, 1, 0)
    return Phi.at[0, 0, 0].set(0.0)


# speedup vs baseline: 19.1899x; 1.0004x over previous
import functools

import numpy as np
import jax
import jax.numpy as jnp
from jax import lax
from jax.experimental import pallas as pl
from jax.experimental.pallas import tpu as pltpu
from jax.experimental.pallas import tpu_sc as plsc

_RES = 128
_SIG = 10.0
_ROWS = 8320
_RCHUNK = 320


def _spec_consts():
    freqs = [np.fft.fftfreq(_RES, d=1.0 / _RES)] * 2
    freqs.append(np.fft.rfftfreq(_RES, d=1.0 / _RES))
    om = np.stack(np.meshgrid(*freqs, indexing="ij"), axis=-1)
    dis = np.sqrt((om ** 2).sum(-1))
    g = np.exp(-0.5 * ((_SIG * 2.0 * dis / _RES) ** 2))
    lap = -np.sum((2.0 * np.pi * om) ** 2, axis=-1)
    c = 2.0 * np.pi * g / (lap + 1e-6)
    b = np.moveaxis(om, -1, 0) * c
    return b.astype(np.float32).reshape(3, _ROWS, 128)


_B_CONST = _spec_consts()


def _spectral_combine(Fr, Fi):
    B = jnp.asarray(_B_CONST)

    def body(fr_ref, fi_ref, b_ref, o_ref):
        b0, b1, b2 = b_ref[0], b_ref[1], b_ref[2]
        for b in range(2):
            o_ref[2 * b] = (b0 * fi_ref[3 * b] + b1 * fi_ref[3 * b + 1]
                            + b2 * fi_ref[3 * b + 2])
            o_ref[2 * b + 1] = -(b0 * fr_ref[3 * b] + b1 * fr_ref[3 * b + 1]
                                 + b2 * fr_ref[3 * b + 2])

    return pl.pallas_call(
        body,
        grid=(_ROWS // _RCHUNK,),
        in_specs=[
            pl.BlockSpec((6, _RCHUNK, 128), lambda i: (0, i, 0)),
            pl.BlockSpec((6, _RCHUNK, 128), lambda i: (0, i, 0)),
            pl.BlockSpec((3, _RCHUNK, 128), lambda i: (0, i, 0)),
        ],
        out_specs=pl.BlockSpec((4, _RCHUNK, 128), lambda i: (0, i, 0)),
        out_shape=jax.ShapeDtypeStruct((4, _ROWS, 128), jnp.float32),
    )(Fr, Fi, B)


_P = 100352
_TPTS = _P // 16
_CHUNK = 1568
_NROW = _CHUNK // 16
_SLABW = 64
_SLAB = _SLABW * _RES * _RES
_DUMP = _SLAB
_ACC = _SLAB + 16384
_STRIPE = _SLAB // 16


def _sc_rasterize(Vt, Nt):
    mesh = plsc.VectorSubcoreMesh(core_axis_name="c", subcore_axis_name="s")

    @functools.partial(
        pl.kernel,
        out_type=jax.ShapeDtypeStruct((12 * _SLAB,), jnp.float32),
        mesh=mesh,
        scratch_types=[
            pltpu.VMEM_SHARED((_ACC,), jnp.float32),
            pltpu.VMEM((2048,), jnp.float32),
            pltpu.VMEM((_CHUNK,), jnp.float32),
            pltpu.VMEM((_CHUNK,), jnp.float32),
            pltpu.VMEM((_CHUNK,), jnp.float32),
            pltpu.VMEM((_CHUNK,), jnp.float32),
            pltpu.VMEM((_NROW * 128,), jnp.int32),
            pltpu.VMEM((_NROW * 128,), jnp.float32),
        ],
        compiler_params=pltpu.CompilerParams(use_tc_tiling_on_sc=True),
    )
    def k(v_hbm, n_hbm, out_hbm, acc, zbuf, px, py, pz, nv, idxb, valb):
        c = lax.axis_index("c")
        s = lax.axis_index("s")

        def zinit(i, carry):
            zbuf[pl.ds(i * 16, 16)] = jnp.zeros((16,), jnp.float32)
            return carry

        lax.fori_loop(0, 2048 // 16, zinit, 0)

        def axis_terms(pref, base, slab):
            t = pref[pl.ds(base, 16)] * 128.0
            i0 = t.astype(jnp.int32)
            f = t - i0.astype(jnp.float32)
            i1 = jnp.where(f > 0.0, i0 + 1, i0) & 127
            return i0, i1, f, slab

        def job(j, carry):
            g = j * 2 + c
            b = g // 6
            f_feat = (g % 6) // 2
            slab = c
            bf = b * 3 + f_feat

            def zero(i, carry2):
                pltpu.sync_copy(
                    zbuf, acc.at[pl.ds(s * _STRIPE + i * 2048, 2048)])
                return carry2

            lax.fori_loop(0, _STRIPE // 2048, zero, 0)
            plsc.subcore_barrier()

            def chunk(cc, carry2):
                pbase = s * _TPTS + cc * _CHUNK
                vb = b * 3 * _P
                pltpu.sync_copy(v_hbm.at[pl.ds(vb + pbase, _CHUNK)], px)
                pltpu.sync_copy(v_hbm.at[pl.ds(vb + _P + pbase, _CHUNK)], py)
                pltpu.sync_copy(v_hbm.at[pl.ds(vb + 2 * _P + pbase, _CHUNK)], pz)
                pltpu.sync_copy(n_hbm.at[pl.ds((b * 3 + f_feat) * _P + pbase, _CHUNK)], nv)

                def row(i, carry3):
                    base = i * 16
                    tx = px[pl.ds(base, 16)] * 128.0
                    x0 = tx.astype(jnp.int32)
                    fx = tx - x0.astype(jnp.float32)
                    x1 = jnp.where(fx > 0.0, x0 + 1, x0) & 127
                    ty = py[pl.ds(base, 16)] * 128.0
                    y0 = ty.astype(jnp.int32)
                    fy = ty - y0.astype(jnp.float32)
                    y1 = jnp.where(fy > 0.0, y0 + 1, y0) & 127
                    tz = pz[pl.ds(base, 16)] * 128.0
                    z0 = tz.astype(jnp.int32)
                    fz = tz - z0.astype(jnp.float32)
                    z1 = jnp.where(fz > 0.0, z0 + 1, z0) & 127
                    val = nv[pl.ds(base, 16)]

                    dump = jnp.full((16,), _DUMP, jnp.int32)
                    xo0 = jnp.where((x0 >> 6) == slab, (x0 & 63) * 16384, dump)
                    xo1 = jnp.where((x1 >> 6) == slab, (x1 & 63) * 16384, dump)
                    a00 = xo0 + y0 * 128
                    a01 = xo0 + y1 * 128
                    a10 = xo1 + y0 * 128
                    a11 = xo1 + y1 * 128
                    wx0 = 1.0 - fx
                    wy0 = 1.0 - fy
                    wz0 = (1.0 - fz) * val
                    wz1 = fz * val
                    w00 = wx0 * wy0
                    w01 = wx0 * fy
                    w10 = fx * wy0
                    w11 = fx * fy
                    idxb[pl.ds(i * 128 + 0, 16)] = a00 + z0
                    valb[pl.ds(i * 128 + 0, 16)] = w00 * wz0
                    idxb[pl.ds(i * 128 + 16, 16)] = a00 + z1
                    valb[pl.ds(i * 128 + 16, 16)] = w00 * wz1
                    idxb[pl.ds(i * 128 + 32, 16)] = a01 + z0
                    valb[pl.ds(i * 128 + 32, 16)] = w01 * wz0
                    idxb[pl.ds(i * 128 + 48, 16)] = a01 + z1
                    valb[pl.ds(i * 128 + 48, 16)] = w01 * wz1
                    idxb[pl.ds(i * 128 + 64, 16)] = a10 + z0
                    valb[pl.ds(i * 128 + 64, 16)] = w10 * wz0
                    idxb[pl.ds(i * 128 + 80, 16)] = a10 + z1
                    valb[pl.ds(i * 128 + 80, 16)] = w10 * wz1
                    idxb[pl.ds(i * 128 + 96, 16)] = a11 + z0
                    valb[pl.ds(i * 128 + 96, 16)] = w11 * wz0
                    idxb[pl.ds(i * 128 + 112, 16)] = a11 + z1
                    valb[pl.ds(i * 128 + 112, 16)] = w11 * wz1
                    return carry3

                lax.fori_loop(0, _NROW, row, 0)
                pltpu.sync_copy(valb, acc.at[idxb], add=True)
                return carry2

            lax.fori_loop(0, _TPTS // _CHUNK, chunk, 0)
            plsc.subcore_barrier()

            pltpu.sync_copy(
                acc.at[pl.ds(s * _STRIPE, _STRIPE)],
                out_hbm.at[pl.ds(bf * 2 * _SLAB + slab * _SLAB + s * _STRIPE,
                                 _STRIPE)])
            return carry

        lax.fori_loop(0, 6, job, 0)

    return k(Vt, Nt)


def _rasterize(V, N):
    Vt = jnp.pad(jnp.transpose(V, (0, 2, 1)),
                 ((0, 0), (0, 0), (0, _P - V.shape[1]))).reshape(-1)
    Nt = jnp.pad(jnp.transpose(N, (0, 2, 1)),
                 ((0, 0), (0, 0), (0, _P - N.shape[1]))).reshape(-1)
    out = _sc_rasterize(Vt, Nt)
    return out.reshape(2, 3, _RES, _RES, _RES)


def kernel(V, N):
    grid = _rasterize(V, N)
    F = jnp.fft.rfftn(grid, axes=(2, 3, 4))
    Fr = jnp.real(F).reshape(6, _ROWS, 128)
    Fi = jnp.imag(F).reshape(6, _ROWS, 128)
    O = _spectral_combine(Fr, Fi)
    Phi = O.reshape(2, 2, 128, 128, 65).transpose(2, 3, 4, 1, 0)
    return Phi.at[0, 0, 0].set(0.0)
